# Initial kernel scaffold; baseline (speedup 1.0000x reference)
import jax
import jax.numpy as jnp
from jax.experimental import pallas as pl


def kernel(input_vector, importance_score, ltm_matrix, ltm_strengths):
    raise NotImplementedError("write your pallas kernel here")


# speedup vs baseline: 2.8739x; 2.8739x over previous
import jax
import jax.numpy as jnp
from jax.experimental import pallas as pl

LTM_SLOTS = 100
VECTOR_DIM = 64
DECAY_RATE = 0.995
IMPORTANCE_THRESHOLD = 0.45
SIMILARITY_THRESHOLD = 0.85
OLD_WEIGHT = 0.8
NEW_WEIGHT = 0.2
BOOST_FACTOR = 0.5


def _nrm(x, eps):
    n = jnp.sqrt(jnp.sum(x * x))
    return x / jnp.maximum(n, eps)


def _tc_body(iv_ref, imp_ref, ltm_ref, str_ref, outm_ref, outs_ref):
    v = iv_ref[...]
    imp = imp_ref[0, 0]
    ltm = ltm_ref[...]
    s = str_ref[...]

    v1 = _nrm(v, 1e-12)
    vn = _nrm(v1, 1e-12)

    norms = jnp.sqrt(jnp.sum(ltm * ltm, axis=1, keepdims=True))
    all_empty = jnp.all(norms < 1e-6)
    ltm_n = ltm / jnp.clip(norms, 1e-8)
    sims = jnp.sum(ltm_n * vn, axis=1, keepdims=True)

    max_sim = jnp.max(sims)
    row_ids = jax.lax.broadcasted_iota(jnp.int32, (LTM_SLOTS, 1), 0)
    msi = jnp.min(jnp.where(sims == max_sim, row_ids, LTM_SLOTS))

    min_s = jnp.min(s)
    col_ids = jax.lax.broadcasted_iota(jnp.int32, (1, LTM_SLOTS), 1)
    wsi = jnp.min(jnp.where(s == min_s, col_ids, LTM_SLOTS))

    reinforce = jnp.logical_and(jnp.logical_not(all_empty),
                                max_sim > SIMILARITY_THRESHOLD)
    slot = jnp.where(reinforce, msi, wsi)

    old_vec = jnp.sum(jnp.where(row_ids == msi, ltm, 0.0), axis=0,
                      keepdims=True)
    merged = _nrm(OLD_WEIGHT * old_vec + NEW_WEIGHT * v1, 1e-12)
    new_vec = jnp.where(reinforce, merged, v1)

    str_msi = jnp.sum(jnp.where(col_ids == msi, s, 0.0))
    boosted = jnp.minimum(str_msi + imp * BOOST_FACTOR, 1.0)
    new_str = jnp.where(reinforce, boosted, imp)

    store = imp > IMPORTANCE_THRESHOLD
    write_row = jnp.logical_and(store, row_ids == slot)
    outm_ref[...] = jnp.where(write_row, new_vec, ltm)

    s2 = jnp.where(jnp.logical_and(store, col_ids == slot), new_str, s)
    s2 = s2 * DECAY_RATE
    outs_ref[...] = s2 * (s2 > 0.01).astype(jnp.float32)


def kernel(input_vector, importance_score, ltm_matrix, ltm_strengths):
    iv = input_vector.reshape(1, VECTOR_DIM)
    imp = importance_score.reshape(1, 1).astype(jnp.float32)
    s = ltm_strengths.reshape(1, LTM_SLOTS)
    outm, outs = pl.pallas_call(
        _tc_body,
        out_shape=(
            jax.ShapeDtypeStruct((LTM_SLOTS, VECTOR_DIM), jnp.float32),
            jax.ShapeDtypeStruct((1, LTM_SLOTS), jnp.float32),
        ),
    )(iv, imp, ltm_matrix, s)
    return outm, outs.reshape(LTM_SLOTS)
